# Initial kernel scaffold; baseline (speedup 1.0000x reference)
import jax
import jax.numpy as jnp
from jax.experimental import pallas as pl


def kernel(x, pcd):
    raise NotImplementedError("write your pallas kernel here")


# speedup vs baseline: 2.4955x; 2.4955x over previous
import functools

import jax
import jax.numpy as jnp
from jax.experimental import pallas as pl

_RADIUS = 0.02
_N_TILE = 2048


def _csdf_kernel(nb, x_ref, p_ref, out_ref):
    j = pl.program_id(1)
    q = x_ref[0]
    p = p_ref[...]
    d2 = None
    for c in range(3):
        diff = q[:, c:c + 1] - p[c:c + 1, :]
        sq = diff * diff
        d2 = sq if d2 is None else d2 + sq
    m = jnp.min(d2)

    @pl.when(j == 0)
    def _():
        out_ref[...] = jnp.full(out_ref.shape, m, dtype=out_ref.dtype)

    @pl.when(j > 0)
    def _():
        out_ref[...] = jnp.minimum(out_ref[...], m)

    @pl.when(j == nb - 1)
    def _():
        out_ref[...] = jnp.sqrt(jnp.maximum(out_ref[...], 0.0)) - _RADIUS


def kernel(x, pcd):
    B, P, _ = x.shape
    N = pcd.shape[0]
    n_pad = -N % _N_TILE
    pcd_t = jnp.pad(pcd, ((0, n_pad), (0, 0)), mode="edge").T
    nb = (N + n_pad) // _N_TILE
    out = pl.pallas_call(
        functools.partial(_csdf_kernel, nb),
        grid=(B, nb),
        in_specs=[
            pl.BlockSpec((1, P, 3), lambda b, j: (b, 0, 0)),
            pl.BlockSpec((3, _N_TILE), lambda b, j: (0, j)),
        ],
        out_specs=pl.BlockSpec((1, 1, 128), lambda b, j: (b, 0, 0)),
        out_shape=jax.ShapeDtypeStruct((B, 1, 128), jnp.float32),
    )(x, pcd_t)
    return out[:, 0, 0]


# speedup vs baseline: 7.9859x; 3.2001x over previous
import functools

import jax
import jax.numpy as jnp
from jax.experimental import pallas as pl
from jax.experimental.pallas import tpu as pltpu

_RADIUS = 0.02
_N_TILE = 2048
_Q_SUB = 128


def _csdf_kernel(nb, nsub, qzlo_ref, qzhi_ref, pzlo_ref, pzhi_ref,
                 order_ref, x_ref, p_ref, out_ref, m_ref):
    b = pl.program_id(0)
    j = pl.program_id(1)

    @pl.when(j == 0)
    def _():
        m_ref[0] = jnp.inf

    p = p_ref[...]
    q = x_ref[0]
    pzl = pzlo_ref[j]
    pzh = pzhi_ref[j]

    for k in range(nsub):
        qzl = qzlo_ref[b, k]
        qzh = qzhi_ref[b, k]
        gap = jnp.maximum(jnp.maximum(pzl - qzh, qzl - pzh), 0.0)

        @pl.when(gap * gap < m_ref[0])
        def _():
            qs = q[k * _Q_SUB:(k + 1) * _Q_SUB, :]
            d2 = None
            for c in range(3):
                diff = qs[:, c:c + 1] - p[c:c + 1, :]
                sq = diff * diff
                d2 = sq if d2 is None else d2 + sq
            m_ref[0] = jnp.minimum(m_ref[0], jnp.min(d2))

    @pl.when(j == nb - 1)
    def _():
        out_ref[...] = jnp.full(
            out_ref.shape,
            jnp.sqrt(jnp.maximum(m_ref[0], 0.0)) - _RADIUS,
            dtype=out_ref.dtype)


def kernel(x, pcd):
    B, P, _ = x.shape
    N = pcd.shape[0]
    n_pad = -N % _N_TILE
    nb = (N + n_pad) // _N_TILE
    nsub = P // _Q_SUB

    ps = jax.lax.sort([pcd[:, 2], pcd[:, 0], pcd[:, 1]], num_keys=1)
    pcd_t = jnp.stack([ps[1], ps[2], ps[0]])
    pcd_t = jnp.pad(pcd_t, ((0, 0), (0, n_pad)), mode="edge")
    qs = jax.lax.sort([x[:, :, 2], x[:, :, 0], x[:, :, 1]],
                      dimension=1, num_keys=1)
    x_s = jnp.stack([qs[1], qs[2], qs[0]], axis=-1)

    pz = pcd_t[2]
    pzlo = pz[::_N_TILE]
    pzhi = pz[_N_TILE - 1::_N_TILE]
    qz = x_s[:, :, 2]
    qzlo = qz[:, ::_Q_SUB]
    qzhi = qz[:, _Q_SUB - 1::_Q_SUB]

    mid = nb // 2
    order = []
    for d in range(nb):
        lo, hi = mid - 1 - d // 2, mid + d // 2
        order.append(hi if d % 2 == 0 else lo)
    order = tuple(order)

    grid_spec = pltpu.PrefetchScalarGridSpec(
        num_scalar_prefetch=5,
        grid=(B, nb),
        in_specs=[
            pl.BlockSpec((1, P, 3), lambda b, j, *_: (b, 0, 0)),
            pl.BlockSpec((3, _N_TILE), lambda b, j, *refs: (0, refs[4][j])),
        ],
        out_specs=pl.BlockSpec((1, 1, 128), lambda b, j, *_: (b, 0, 0)),
        scratch_shapes=[pltpu.SMEM((1,), jnp.float32)],
    )
    out = pl.pallas_call(
        functools.partial(_csdf_kernel, nb, nsub),
        grid_spec=grid_spec,
        out_shape=jax.ShapeDtypeStruct((B, 1, 128), jnp.float32),
    )(qzlo, qzhi, pzlo[jnp.array(order)], pzhi[jnp.array(order)],
      jnp.array(order, dtype=jnp.int32), x_s, pcd_t)
    return out[:, 0, 0]


# speedup vs baseline: 8.4560x; 1.0589x over previous
import functools

import jax
import jax.numpy as jnp
from jax.experimental import pallas as pl
from jax.experimental.pallas import tpu as pltpu

_RADIUS = 0.02
_N_TILE = 2048
_Q_SUB = 128


def _csdf_kernel(nb, nsub, qzlo_ref, qzhi_ref, pzlo_ref, pzhi_ref,
                 x_ref, p_ref, out_ref, m_ref):
    b = pl.program_id(0)
    m_ref[0] = jnp.inf
    q = x_ref[0]

    def tile_body(j, carry):
        pt = p_ref[j]
        pzl = pzlo_ref[j]
        pzh = pzhi_ref[j]
        for k in range(nsub):
            qzl = qzlo_ref[b, k]
            qzh = qzhi_ref[b, k]
            gap = jnp.maximum(jnp.maximum(pzl - qzh, qzl - pzh), 0.0)

            @pl.when(gap * gap < m_ref[0])
            def _():
                qs = q[k * _Q_SUB:(k + 1) * _Q_SUB, :]
                d2 = None
                for c in range(3):
                    diff = qs[:, c:c + 1] - pt[c:c + 1, :]
                    sq = diff * diff
                    d2 = sq if d2 is None else d2 + sq
                m_ref[0] = jnp.minimum(m_ref[0], jnp.min(d2))
        return carry

    jax.lax.fori_loop(0, nb, tile_body, 0)

    out_ref[...] = jnp.full(
        out_ref.shape,
        jnp.sqrt(jnp.maximum(m_ref[0], 0.0)) - _RADIUS,
        dtype=out_ref.dtype)


def kernel(x, pcd):
    B, P, _ = x.shape
    N = pcd.shape[0]
    n_pad = -N % _N_TILE
    nb = (N + n_pad) // _N_TILE
    nsub = P // _Q_SUB

    ps = jax.lax.sort([pcd[:, 2], pcd[:, 0], pcd[:, 1]], num_keys=1)
    pcd_t = jnp.stack([ps[1], ps[2], ps[0]])
    pcd_t = jnp.pad(pcd_t, ((0, 0), (0, n_pad)), mode="edge")
    qs = jax.lax.sort([x[:, :, 2], x[:, :, 0], x[:, :, 1]],
                      dimension=1, num_keys=1)
    x_s = jnp.stack([qs[1], qs[2], qs[0]], axis=-1)

    pz = pcd_t[2]
    pzlo = pz[::_N_TILE]
    pzhi = pz[_N_TILE - 1::_N_TILE]
    qz = x_s[:, :, 2]
    qzlo = qz[:, ::_Q_SUB]
    qzhi = qz[:, _Q_SUB - 1::_Q_SUB]

    mid = nb // 2
    order = []
    for d in range(nb):
        lo, hi = mid - 1 - d // 2, mid + d // 2
        order.append(hi if d % 2 == 0 else lo)
    order = jnp.array(order, dtype=jnp.int32)
    p_tiles = pcd_t.reshape(3, nb, _N_TILE).transpose(1, 0, 2)
    p_tiles = p_tiles[order]

    grid_spec = pltpu.PrefetchScalarGridSpec(
        num_scalar_prefetch=4,
        grid=(B,),
        in_specs=[
            pl.BlockSpec((1, P, 3), lambda b, *_: (b, 0, 0)),
            pl.BlockSpec((nb, 3, _N_TILE), lambda b, *_: (0, 0, 0)),
        ],
        out_specs=pl.BlockSpec((1, 1, 128), lambda b, *_: (b, 0, 0)),
        scratch_shapes=[pltpu.SMEM((1,), jnp.float32)],
    )
    out = pl.pallas_call(
        functools.partial(_csdf_kernel, nb, nsub),
        grid_spec=grid_spec,
        out_shape=jax.ShapeDtypeStruct((B, 1, 128), jnp.float32),
    )(qzlo, qzhi, pzlo[order], pzhi[order], x_s, p_tiles)
    return out[:, 0, 0]
